# Initial kernel scaffold; baseline (speedup 1.0000x reference)
import jax
import jax.numpy as jnp
from jax.experimental import pallas as pl


def kernel(x, mask):
    raise NotImplementedError("write your pallas kernel here")


# speedup vs baseline: 1.0284x; 1.0284x over previous
import jax
import jax.numpy as jnp
from jax.experimental import pallas as pl
from jax.experimental.pallas import tpu as pltpu

_B, _C, _H, _W = 4, 96, 384, 384
_HT = 64


def _pool_body(x_ref, m_ref, sums_ref, cnt_ref):
    b = pl.program_id(0)
    h = pl.program_id(1)

    @pl.when((b == 0) & (h == 0))
    def _init():
        sums_ref[...] = jnp.zeros_like(sums_ref)
        cnt_ref[0, 0] = jnp.float32(0.0)

    xb = x_ref[0]
    mb = m_ref[0].astype(jnp.float32)
    s1 = jnp.sum(xb * mb[None, :, :], axis=(1, 2))
    s0 = jnp.sum(xb, axis=(1, 2))
    sums_ref[...] += jnp.stack([s1, s0])
    cnt_ref[0, 0] += jnp.sum(mb)


def kernel(x, mask):
    B, C, H, W = x.shape
    grid = (B, H // _HT)
    sums, cnt = pl.pallas_call(
        _pool_body,
        grid=grid,
        in_specs=[
            pl.BlockSpec((1, C, _HT, W), lambda b, h: (b, 0, h, 0)),
            pl.BlockSpec((1, _HT, W), lambda b, h: (b, h, 0)),
        ],
        out_specs=[
            pl.BlockSpec((2, C), lambda b, h: (0, 0)),
            pl.BlockSpec(memory_space=pltpu.SMEM),
        ],
        out_shape=[
            jax.ShapeDtypeStruct((2, C), jnp.float32),
            jax.ShapeDtypeStruct((1, 1), jnp.float32),
        ],
    )(x, mask)
    n_ch = cnt[0, 0]
    n_tot = jnp.float32(B * H * W)
    ch = sums[0] / n_ch
    unch = (sums[1] - sums[0]) / (n_tot - n_ch)
    return (unch, ch)
